# baseline (speedup 1.0000x reference)
import functools
import math

import jax
import jax.numpy as jnp
from jax import lax
from jax.experimental import pallas as pl
from jax.experimental.pallas import tpu as pltpu
from jax.experimental.pallas import tpu_sc as plsc


def _make_sc_kernel(B, L, V, D, scale):
    try:
        info = plsc.get_sparse_core_info()
        NC, NS, LANES = info.num_cores, info.num_subcores, info.num_lanes
    except ValueError:
        NC, NS, LANES = 2, 16, 16
    NW = NC * NS
    BG = B // 128
    LQ = NW // BG
    LPW = L // LQ
    DG = D // 8
    assert BG * LQ == NW and LPW * LQ == L and DG * 8 == D and D % LANES == 0
    NBUF = 2
    UNROLL = 4
    PSTRIDE = D + 1

    mesh = plsc.VectorSubcoreMesh(
        core_axis_name="c", subcore_axis_name="s", num_cores=NC, num_subcores=NS)

    @functools.partial(
        pl.kernel,
        out_type=jax.ShapeDtypeStruct((L, DG, BG, 8, 128), jnp.float32),
        mesh=mesh,
        compiler_params=pltpu.CompilerParams(
            use_tc_tiling_on_sc=False, needs_layout_passes=False),
        scratch_types=[
            pltpu.VMEM((LPW, 128), jnp.int32),
            pltpu.VMEM((LPW * D,), jnp.float32),
            pltpu.VMEM((NBUF, 128, D), jnp.float32),
            pltpu.VMEM((128 * PSTRIDE,), jnp.float32),
            pltpu.VMEM((NBUF, DG, 8, 128), jnp.float32),
            pltpu.SemaphoreType.DMA,
            pltpu.SemaphoreType.DMA,
            pltpu.SemaphoreType.DMA,
            pltpu.SemaphoreType.DMA,
        ],
    )
    def k(tok_hbm, idx_hbm, pos_hbm, out_hbm, idx_v, pos_v, g_v, p_v, o_v,
          gsem0, gsem1, osem0, osem1):
        wid = lax.axis_index("s") * NC + lax.axis_index("c")
        bg = wid % BG
        l0 = (wid // BG) * LPW
        gsem = (gsem0, gsem1)
        osem = (osem0, osem1)

        pltpu.sync_copy(idx_hbm.at[pl.ds(l0, LPW), bg], idx_v)
        pltpu.sync_copy(pos_hbm.at[pl.ds(l0 * D, LPW * D)], pos_v)

        iota = jnp.arange(16, dtype=jnp.int32)
        tbase = [(iota + 16 * kk) * PSTRIDE for kk in range(8)]

        def start_gather(lr, b):
            return pltpu.async_copy(
                tok_hbm.at[idx_v.at[lr]], g_v.at[b], gsem[b])

        def wait_gather(lr, b):
            pltpu.make_async_copy(
                tok_hbm.at[idx_v.at[lr]], g_v.at[b], gsem[b]).wait()

        def start_out(l_abs, b):
            return pltpu.async_copy(
                o_v.at[b], out_hbm.at[l_abs, :, bg], osem[b])

        def wait_out(l_abs, b):
            pltpu.make_async_copy(
                o_v.at[b], out_hbm.at[l_abs, :, bg], osem[b]).wait()

        def compute(lr, b):
            pchunks = [pos_v[pl.ds(lr * D + j * 16, 16)] for j in range(D // 16)]

            @plsc.parallel_loop(0, 128, unroll=UNROLL)
            def _(bb):
                for j in range(D // 16):
                    p_v[pl.ds(bb * PSTRIDE + j * 16, 16)] = (
                        g_v[b, bb, pl.ds(j * 16, 16)] * scale + pchunks[j])

            @plsc.parallel_loop(0, D, unroll=UNROLL)
            def _(d):
                dg_i = lax.shift_right_logical(d, 3)
                di_i = lax.bitwise_and(d, 7)
                for kk in range(8):
                    vals = plsc.load_gather(p_v, [tbase[kk] + d])
                    o_v[b, dg_i, di_i, pl.ds(kk * 16, 16)] = vals

        for b in range(NBUF):
            start_gather(b, b)

        for b in range(NBUF):
            wait_gather(b, b)
            compute(b, b)
            start_out(l0 + b, b)
            start_gather(b + NBUF, b)

        def round_body(r0, carry):
            for b in range(NBUF):
                lr = r0 * NBUF + b
                wait_gather(lr, b)
                wait_out(l0 + lr - NBUF, b)
                compute(lr, b)
                start_out(l0 + lr, b)

                @pl.when(lr + NBUF < LPW)
                def _():
                    start_gather(lr + NBUF, b)

            return carry

        lax.fori_loop(1, LPW // NBUF, round_body, 0)

        for b in range(NBUF):
            wait_out(l0 + LPW - NBUF + b, b)

    return k


def kernel(inputs, token_table, pos_table):
    B, L = inputs.shape
    V, D = token_table.shape
    scale = float(math.sqrt(D))
    idx_t = jnp.transpose(inputs.astype(jnp.int32)).reshape(L, B // 128, 128)
    pos_flat = pos_table.reshape(L * D)
    k = _make_sc_kernel(B, L, V, D, scale)
    kout = k(token_table, idx_t, pos_flat)
    return kout.transpose((2, 4, 0, 1, 3)).reshape(B, L, D)


# speedup vs baseline: 1.0799x; 1.0799x over previous
import functools
import math

import jax
import jax.numpy as jnp
from jax import lax
from jax.experimental import pallas as pl
from jax.experimental.pallas import tpu as pltpu
from jax.experimental.pallas import tpu_sc as plsc


def _make_sc_kernel(B, L, V, D, scale):
    try:
        info = plsc.get_sparse_core_info()
        NC, NS, LANES = info.num_cores, info.num_subcores, info.num_lanes
    except ValueError:
        NC, NS, LANES = 2, 16, 16
    NW = NC * NS
    BG = B // 128
    LQ = NW // BG
    LPW = L // LQ
    DG = D // 8
    assert BG * LQ == NW and LPW * LQ == L and DG * 8 == D and D % LANES == 0
    NBUF = 5
    UNROLL = 4
    PSTRIDE = D + 1

    mesh = plsc.VectorSubcoreMesh(
        core_axis_name="c", subcore_axis_name="s", num_cores=NC, num_subcores=NS)

    @functools.partial(
        pl.kernel,
        out_type=jax.ShapeDtypeStruct((L, DG, BG, 8, 128), jnp.float32),
        mesh=mesh,
        compiler_params=pltpu.CompilerParams(
            use_tc_tiling_on_sc=False, needs_layout_passes=False),
        scratch_types=[
            pltpu.VMEM((LPW, 128), jnp.int32),
            pltpu.VMEM((LPW * D,), jnp.float32),
            pltpu.VMEM((NBUF, 128, D), jnp.float32),
            pltpu.VMEM((128 * PSTRIDE,), jnp.float32),
            pltpu.VMEM((NBUF, DG, 8, 128), jnp.float32),
        ] + [pltpu.SemaphoreType.DMA] * (2 * NBUF),
    )
    def k(tok_hbm, idx_hbm, pos_hbm, out_hbm, idx_v, pos_v, g_v, p_v, o_v,
          *sems):
        wid = lax.axis_index("s") * NC + lax.axis_index("c")
        bg = wid % BG
        l0 = (wid // BG) * LPW
        gsem = sems[:NBUF]
        osem = sems[NBUF:]

        pltpu.sync_copy(idx_hbm.at[pl.ds(l0, LPW), bg], idx_v)
        pltpu.sync_copy(pos_hbm.at[pl.ds(l0 * D, LPW * D)], pos_v)

        iota = jnp.arange(16, dtype=jnp.int32)
        tbase = [(iota + 16 * kk) * PSTRIDE for kk in range(8)]

        def start_gather(lr, b):
            return pltpu.async_copy(
                tok_hbm.at[idx_v.at[lr]], g_v.at[b], gsem[b])

        def wait_gather(lr, b):
            pltpu.make_async_copy(
                tok_hbm.at[idx_v.at[lr]], g_v.at[b], gsem[b]).wait()

        def start_out(l_abs, b):
            return pltpu.async_copy(
                o_v.at[b], out_hbm.at[l_abs, :, bg], osem[b])

        def wait_out(l_abs, b):
            pltpu.make_async_copy(
                o_v.at[b], out_hbm.at[l_abs, :, bg], osem[b]).wait()

        def compute(lr, b):
            pchunks = [pos_v[pl.ds(lr * D + j * 16, 16)] for j in range(D // 16)]

            @plsc.parallel_loop(0, 128, unroll=UNROLL)
            def _(bb):
                for j in range(D // 16):
                    p_v[pl.ds(bb * PSTRIDE + j * 16, 16)] = (
                        g_v[b, bb, pl.ds(j * 16, 16)] * scale + pchunks[j])

            @plsc.parallel_loop(0, D, unroll=UNROLL)
            def _(d):
                dg_i = lax.shift_right_logical(d, 3)
                di_i = lax.bitwise_and(d, 7)
                for kk in range(8):
                    vals = plsc.load_gather(p_v, [tbase[kk] + d])
                    o_v[b, dg_i, di_i, pl.ds(kk * 16, 16)] = vals

        for b in range(NBUF):
            start_gather(b, b)

        for b in range(NBUF):
            wait_gather(b, b)
            compute(b, b)
            start_out(l0 + b, b)
            start_gather(b + NBUF, b)

        def round_body(r0, carry):
            for b in range(NBUF):
                lr = r0 * NBUF + b
                wait_gather(lr, b)
                wait_out(l0 + lr - NBUF, b)
                compute(lr, b)
                start_out(l0 + lr, b)

                @pl.when(lr + NBUF < LPW)
                def _():
                    start_gather(lr + NBUF, b)

            return carry

        lax.fori_loop(1, LPW // NBUF, round_body, 0)

        for b in range(NBUF):
            wait_out(l0 + LPW - NBUF + b, b)

    return k


def kernel(inputs, token_table, pos_table):
    B, L = inputs.shape
    V, D = token_table.shape
    scale = float(math.sqrt(D))
    idx_t = jnp.transpose(inputs.astype(jnp.int32)).reshape(L, B // 128, 128)
    pos_flat = pos_table.reshape(L * D)
    k = _make_sc_kernel(B, L, V, D, scale)
    kout = k(token_table, idx_t, pos_flat)
    return kout.transpose((2, 4, 0, 1, 3)).reshape(B, L, D)


# speedup vs baseline: 1.0850x; 1.0047x over previous
import functools
import math

import jax
import jax.numpy as jnp
from jax import lax
from jax.experimental import pallas as pl
from jax.experimental.pallas import tpu as pltpu
from jax.experimental.pallas import tpu_sc as plsc


def _make_sc_kernel(B, L, V, D, scale):
    try:
        info = plsc.get_sparse_core_info()
        NC, NS, LANES = info.num_cores, info.num_subcores, info.num_lanes
    except ValueError:
        NC, NS, LANES = 2, 16, 16
    NW = NC * NS
    BG = B // 128
    LQ = NW // BG
    LPW = L // LQ
    DG = D // 8
    assert BG * LQ == NW and LPW * LQ == L and DG * 8 == D and D % LANES == 0
    NBUF = 5
    UNROLL = 4
    PSTRIDE = D + 1

    mesh = plsc.VectorSubcoreMesh(
        core_axis_name="c", subcore_axis_name="s", num_cores=NC, num_subcores=NS)

    @functools.partial(
        pl.kernel,
        out_type=jax.ShapeDtypeStruct((L, DG, BG, 8, 128), jnp.float32),
        mesh=mesh,
        compiler_params=pltpu.CompilerParams(
            use_tc_tiling_on_sc=False, needs_layout_passes=False),
        scratch_types=[
            pltpu.VMEM((LPW, 128), jnp.int32),
            pltpu.VMEM((LPW * D,), jnp.float32),
            pltpu.VMEM((NBUF, 128, D), jnp.float32),
            pltpu.VMEM((128 * PSTRIDE,), jnp.float32),
            pltpu.VMEM((NBUF, DG, 8, 128), jnp.float32),
        ] + [pltpu.SemaphoreType.DMA] * (2 * NBUF),
    )
    def k(tok_hbm, idx_hbm, pos_hbm, out_hbm, idx_v, pos_v, g_v, p_v, o_v,
          *sems):
        wid = lax.axis_index("s") * NC + lax.axis_index("c")
        bg = wid % BG
        l0 = (wid // BG) * LPW
        gsem = sems[:NBUF]
        osem = sems[NBUF:]

        pltpu.sync_copy(idx_hbm.at[pl.ds(l0, LPW), bg], idx_v)
        pltpu.sync_copy(pos_hbm.at[pl.ds(l0 * D, LPW * D)], pos_v)

        iota = jnp.arange(16, dtype=jnp.int32)
        tbase = [(iota + 16 * kk) * PSTRIDE for kk in range(8)]

        def start_gather(lr, b):
            return pltpu.async_copy(
                tok_hbm.at[idx_v.at[lr]], g_v.at[b], gsem[b])

        def wait_gather(lr, b):
            pltpu.make_async_copy(
                tok_hbm.at[idx_v.at[lr]], g_v.at[b], gsem[b]).wait()

        def start_out(l_abs, b):
            return pltpu.async_copy(
                o_v.at[b], out_hbm.at[l_abs, :, bg], osem[b])

        def wait_out(l_abs, b):
            pltpu.make_async_copy(
                o_v.at[b], out_hbm.at[l_abs, :, bg], osem[b]).wait()

        def pass1(lr, b):
            pchunks = [pos_v[pl.ds(lr * D + j * 16, 16)] for j in range(D // 16)]

            @plsc.parallel_loop(0, 128, unroll=UNROLL)
            def _(bb):
                for j in range(D // 16):
                    p_v[pl.ds(bb * PSTRIDE + j * 16, 16)] = (
                        g_v[b, bb, pl.ds(j * 16, 16)] * scale + pchunks[j])

        def pass2(b):
            @plsc.parallel_loop(0, D, unroll=UNROLL)
            def _(d):
                dg_i = lax.shift_right_logical(d, 3)
                di_i = lax.bitwise_and(d, 7)
                for kk in range(8):
                    vals = plsc.load_gather(p_v, [tbase[kk] + d])
                    o_v[b, dg_i, di_i, pl.ds(kk * 16, 16)] = vals

        for b in range(NBUF):
            start_gather(b, b)

        for b in range(NBUF):
            wait_gather(b, b)
            pass1(b, b)
            start_gather(b + NBUF, b)
            pass2(b)
            start_out(l0 + b, b)

        def round_body(r0, carry):
            for b in range(NBUF):
                lr = r0 * NBUF + b
                wait_gather(lr, b)
                pass1(lr, b)

                @pl.when(lr + NBUF < LPW)
                def _():
                    start_gather(lr + NBUF, b)

                wait_out(l0 + lr - NBUF, b)
                pass2(b)
                start_out(l0 + lr, b)

            return carry

        lax.fori_loop(1, LPW // NBUF, round_body, 0)

        for b in range(NBUF):
            wait_out(l0 + LPW - NBUF + b, b)

    return k


def kernel(inputs, token_table, pos_table):
    B, L = inputs.shape
    V, D = token_table.shape
    scale = float(math.sqrt(D))
    idx_t = jnp.transpose(inputs.astype(jnp.int32)).reshape(L, B // 128, 128)
    pos_flat = pos_table.reshape(L * D)
    k = _make_sc_kernel(B, L, V, D, scale)
    kout = k(token_table, idx_t, pos_flat)
    return kout.transpose((2, 4, 0, 1, 3)).reshape(B, L, D)
